# baseline (speedup 1.0000x reference)
import jax
import jax.numpy as jnp
from jax import lax
from jax.experimental import pallas as pl
from jax.experimental.pallas import tpu as pltpu
from jax.experimental.pallas import tpu_sc as plsc

BATCH = 16
PER_IMG = 2048
NSP = 256
NACT = 128
NSP_V = NSP // 16
NACT_V = NACT // 16
NOUT = NSP + NACT
CHUNK = 64
HALF = PER_IMG // 2
NCHUNKS = HALF // CHUNK


def _sc_body(sp_hbm, act_hbm, out_hbm, sbuf, abuf, pbuf, qbuf, shared):
    c = lax.axis_index("c")
    s = lax.axis_index("s")
    img = c * 8 + s // 2
    half = s % 2
    row0 = img * PER_IMG + half * HALF

    sacc = tuple(jnp.zeros((16,), jnp.float32) for _ in range(NSP_V))
    aacc = tuple(jnp.zeros((16,), jnp.float32) for _ in range(NACT_V))

    def row_body(r, accs):
        sacc, aacc = accs
        xs = [sbuf[r, pl.ds(16 * j, 16)] for j in range(NSP_V)]
        m = xs[0]
        for x in xs[1:]:
            m = jnp.maximum(m, x)
        ms = jnp.max(m)
        es = [jnp.exp(x - ms) for x in xs]
        tot = es[0]
        for e in es[1:]:
            tot = tot + e
        rs = 1.0 / lax.broadcast(jnp.sum(tot), (16,))
        sacc = tuple(jnp.maximum(a, e * rs) for a, e in zip(sacc, es))

        ys = [abuf[r, pl.ds(16 * j, 16)] for j in range(NACT_V)]
        ma = ys[0]
        for y in ys[1:]:
            ma = jnp.maximum(ma, y)
        mas = jnp.max(ma)
        fs = [jnp.exp(y - mas) for y in ys]
        tota = fs[0]
        for f in fs[1:]:
            tota = tota + f
        ra = 1.0 / lax.broadcast(jnp.sum(tota), (16,))
        aacc = tuple(jnp.maximum(a, f * ra) for a, f in zip(aacc, fs))
        return (sacc, aacc)

    for k in range(NCHUNKS):
        pltpu.sync_copy(sp_hbm.at[pl.ds(row0 + k * CHUNK, CHUNK)], sbuf)
        pltpu.sync_copy(act_hbm.at[pl.ds(row0 + k * CHUNK, CHUNK)], abuf)
        sacc, aacc = lax.fori_loop(0, CHUNK, row_body, (sacc, aacc))

    for j in range(NSP_V):
        pbuf[pl.ds(16 * j, 16)] = sacc[j]
    for j in range(NACT_V):
        pbuf[pl.ds(NSP + 16 * j, 16)] = aacc[j]

    pltpu.sync_copy(pbuf, shared.at[s])
    plsc.subcore_barrier()

    @pl.when(half == 0)
    def _():
        pltpu.sync_copy(shared.at[s + 1], qbuf)
        for j in range(NOUT // 16):
            d = pl.ds(16 * j, 16)
            pbuf[d] = jnp.maximum(pbuf[d], qbuf[d])
        pltpu.sync_copy(pbuf, out_hbm.at[img])


def kernel(species_logits, activity_logits, box_counts):
    del box_counts
    f = pl.kernel(
        _sc_body,
        out_type=jax.ShapeDtypeStruct((BATCH, NOUT), jnp.float32),
        mesh=plsc.VectorSubcoreMesh(core_axis_name="c", subcore_axis_name="s"),
        compiler_params=pltpu.CompilerParams(needs_layout_passes=False),
        scratch_types=[
            pltpu.VMEM((CHUNK, NSP), jnp.float32),
            pltpu.VMEM((CHUNK, NACT), jnp.float32),
            pltpu.VMEM((NOUT,), jnp.float32),
            pltpu.VMEM((NOUT,), jnp.float32),
            pltpu.VMEM_SHARED((16, NOUT), jnp.float32),
        ],
    )
    return f(species_logits, activity_logits)
